# Initial kernel scaffold; baseline (speedup 1.0000x reference)
import functools

import jax
import jax.numpy as jnp
from jax import lax
from jax.experimental import pallas as pl
from jax.experimental.pallas import tpu as pltpu
from jax.experimental.pallas import tpu_sc as plsc

D = 64
NC = 2
NS = 16
NW = NC * NS
CHUNK = 640
SUB = 128
NSUB = CHUNK // SUB


@functools.partial(jax.jit, static_argnames=("n_chunks",))
def _gather(table, flat_ids, n_chunks):
    bf = flat_ids.shape[0]
    b_per_w = bf // NW
    mesh = plsc.VectorSubcoreMesh(core_axis_name="c", subcore_axis_name="s")

    @functools.partial(
        pl.kernel,
        mesh=mesh,
        out_type=jax.ShapeDtypeStruct((bf, D), jnp.float32),
        scratch_types=[
            pltpu.VMEM((CHUNK,), jnp.int32),
            pltpu.VMEM((CHUNK, D), jnp.float32),
            pltpu.SemaphoreType.DMA,
        ],
    )
    def k(table_hbm, idx_hbm, out_hbm, idx_v, rows_v, sem):
        wid = lax.axis_index("s") * NC + lax.axis_index("c")
        base_w = wid * b_per_w

        def body(i, carry):
            base = base_w + i * CHUNK
            pltpu.sync_copy(idx_hbm.at[pl.ds(base, CHUNK)], idx_v)
            descs = [
                pltpu.async_copy(
                    table_hbm.at[idx_v.at[pl.ds(j * SUB, SUB)]],
                    rows_v.at[pl.ds(j * SUB, SUB)],
                    sem,
                )
                for j in range(NSUB)
            ]
            for dsc in descs:
                dsc.wait()
            pltpu.sync_copy(rows_v, out_hbm.at[pl.ds(base, CHUNK)])
            return carry

        lax.fori_loop(0, n_chunks, body, 0)

    return k(table, flat_ids)


def kernel(ids, table):
    b, h = ids.shape
    flat = ids.reshape(-1).astype(jnp.int32)
    bf = flat.shape[0]
    step = NW * CHUNK
    pad = (-bf) % step
    if pad:
        flat = jnp.concatenate([flat, jnp.zeros((pad,), jnp.int32)])
    n_chunks = (bf + pad) // step
    out = _gather(table, flat, n_chunks)
    if pad:
        out = out[:bf]
    return out.reshape(b, h, D)


# speedup vs baseline: 1.8141x; 1.8141x over previous
import functools

import jax
import jax.numpy as jnp
from jax import lax
from jax.experimental import pallas as pl
from jax.experimental.pallas import tpu as pltpu
from jax.experimental.pallas import tpu_sc as plsc

D = 64
NC = 2
NS = 16
NW = NC * NS
CHUNK = 640
SUB = 128
NSUB = CHUNK // SUB


@functools.partial(jax.jit, static_argnames=("n_chunks",))
def _gather(table, flat_ids, n_chunks):
    bf = flat_ids.shape[0]
    b_per_w = bf // NW
    mesh = plsc.VectorSubcoreMesh(core_axis_name="c", subcore_axis_name="s")

    @functools.partial(
        pl.kernel,
        mesh=mesh,
        out_type=jax.ShapeDtypeStruct((bf, D), jnp.float32),
        scratch_types=[
            pltpu.VMEM((CHUNK,), jnp.int32),
            pltpu.VMEM((CHUNK, D), jnp.float32),
            pltpu.SemaphoreType.DMA,
        ],
        compiler_params=pltpu.CompilerParams(use_tc_tiling_on_sc=False),
    )
    def k(table_hbm, idx_hbm, out_hbm, idx_v, rows_v, sem):
        wid = lax.axis_index("s") * NC + lax.axis_index("c")
        base_w = wid * b_per_w

        def body(i, carry):
            base = base_w + i * CHUNK
            pltpu.sync_copy(idx_hbm.at[pl.ds(base, CHUNK)], idx_v)
            descs = [
                pltpu.async_copy(
                    table_hbm.at[idx_v.at[pl.ds(j * SUB, SUB)]],
                    rows_v.at[pl.ds(j * SUB, SUB)],
                    sem,
                )
                for j in range(NSUB)
            ]
            for dsc in descs:
                dsc.wait()
            pltpu.sync_copy(rows_v, out_hbm.at[pl.ds(base, CHUNK)])
            return carry

        lax.fori_loop(0, n_chunks, body, 0)

    return k(table, flat_ids)


def kernel(ids, table):
    b, h = ids.shape
    flat = ids.reshape(-1).astype(jnp.int32)
    bf = flat.shape[0]
    step = NW * CHUNK
    pad = (-bf) % step
    if pad:
        flat = jnp.concatenate([flat, jnp.zeros((pad,), jnp.int32)])
    n_chunks = (bf + pad) // step
    out = _gather(table, flat, n_chunks)
    if pad:
        out = out[:bf]
    return out.reshape(b, h, D)


# speedup vs baseline: 1.8777x; 1.0351x over previous
import functools

import jax
import jax.numpy as jnp
from jax import lax
from jax.experimental import pallas as pl
from jax.experimental.pallas import tpu as pltpu
from jax.experimental.pallas import tpu_sc as plsc

D = 64
NC = 2
NS = 16
NW = NC * NS
CHUNK = 640
SUB = 128
NSUB = CHUNK // SUB


@functools.partial(jax.jit, static_argnames=("n_chunks",))
def _gather(table, flat_ids, n_chunks):
    bf = flat_ids.shape[0]
    b_per_w = bf // NW
    mesh = plsc.VectorSubcoreMesh(core_axis_name="c", subcore_axis_name="s")

    @functools.partial(
        pl.kernel,
        mesh=mesh,
        out_type=jax.ShapeDtypeStruct((bf, D), jnp.float32),
        scratch_types=[
            pltpu.VMEM((2, CHUNK), jnp.int32),
            pltpu.VMEM((2, CHUNK, D), jnp.float32),
            pltpu.SemaphoreType.DMA,
            pltpu.SemaphoreType.DMA,
            pltpu.SemaphoreType.DMA,
            pltpu.SemaphoreType.DMA,
        ],
        compiler_params=pltpu.CompilerParams(use_tc_tiling_on_sc=False),
    )
    def k(table_hbm, idx_hbm, out_hbm, idx_v, rows_v, g0, g1, o0, o1):
        gsem = (g0, g1)
        osem = (o0, o1)
        wid = lax.axis_index("s") * NC + lax.axis_index("c")
        base_w = wid * b_per_w

        def load_idx(b, g):
            pltpu.sync_copy(idx_hbm.at[pl.ds(base_w + g * CHUNK, CHUNK)],
                            idx_v.at[b])

        def fire_gather(b):
            for j in range(NSUB):
                pltpu.async_copy(
                    table_hbm.at[idx_v.at[b, pl.ds(j * SUB, SUB)]],
                    rows_v.at[b, pl.ds(j * SUB, SUB)],
                    gsem[b],
                )

        def wait_gather(b):
            for j in range(NSUB):
                pltpu.make_async_copy(
                    table_hbm.at[idx_v.at[b, pl.ds(j * SUB, SUB)]],
                    rows_v.at[b, pl.ds(j * SUB, SUB)],
                    gsem[b],
                ).wait()

        def fire_out(b, g):
            pltpu.async_copy(rows_v.at[b],
                             out_hbm.at[pl.ds(base_w + g * CHUNK, CHUNK)],
                             osem[b])

        def wait_out(b):
            pltpu.make_async_copy(rows_v.at[b],
                                  out_hbm.at[pl.ds(base_w, CHUNK)],
                                  osem[b]).wait()


        load_idx(0, 0)
        fire_gather(0)
        load_idx(1, 1)
        fire_gather(1)
        wait_gather(0)
        fire_out(0, 0)

        def body2(kk, carry):
            for p in (1, 0):
                g = 2 * kk + 2 - p
                load_idx(1 - p, g + 1)
                wait_out(1 - p)
                fire_gather(1 - p)
                wait_gather(p)
                fire_out(p, g)
            return carry

        lax.fori_loop(0, (n_chunks - 2) // 2, body2, 0)

        pl_last = (n_chunks - 1) % 2
        wait_gather(pl_last)
        fire_out(pl_last, n_chunks - 1)
        wait_out(1 - pl_last)
        wait_out(pl_last)

    return k(table, flat_ids)


def kernel(ids, table):
    b, h = ids.shape
    flat = ids.reshape(-1).astype(jnp.int32)
    bf = flat.shape[0]
    step = NW * CHUNK * 2
    pad = (-bf) % step
    if pad:
        flat = jnp.concatenate([flat, jnp.zeros((pad,), jnp.int32)])
    n_chunks = (bf + pad) // (NW * CHUNK)
    out = _gather(table, flat, n_chunks)
    if pad:
        out = out[:bf]
    return out.reshape(b, h, D)
